# baseline (speedup 1.0000x reference)
import functools

import jax
import jax.numpy as jnp
from jax import lax
from jax.experimental import pallas as pl
from jax.experimental.pallas import tpu as pltpu
from jax.experimental.pallas import tpu_sc as plsc

_N = 10000
_E = 160000
_U = 128
_CUTOFF = 5.0

_BE = 640
_ESC = _E + 64
_BN = 400
_K = 32
_NPW = 320


def _cut_body(ew_ref, c_ref):
    r = ew_ref[...]
    c_ref[...] = (0.5 * (jnp.cos(jnp.pi * r / _CUTOFF) + 1.0)
                  * (r < _CUTOFF).astype(jnp.float32))


def _cutoff(ew2):
    return pl.pallas_call(
        _cut_body,
        out_shape=jax.ShapeDtypeStruct(ew2.shape, jnp.float32),
    )(ew2)


def _mlp_body(ea_ref, w1t_ref, b1_ref, w2t_ref, b2_ref, w3pt_ref,
              b3p_ref, w_ref):
    ea = ea_ref[...]
    h = jax.nn.silu(jnp.dot(ea, w1t_ref[...],
                            preferred_element_type=jnp.float32) + b1_ref[...])
    h = jax.nn.silu(jnp.dot(h, w2t_ref[...],
                            preferred_element_type=jnp.float32) + b2_ref[...])
    h = jax.nn.silu(jnp.dot(h, w3pt_ref[...],
                            preferred_element_type=jnp.float32) + b3p_ref[...])
    w_ref[...] = h


def _edge_mlp(ea, w1t, b1, w2t, b2, w3pt, b3p):
    grid = (_E // _BE,)
    return pl.pallas_call(
        _mlp_body,
        grid=grid,
        in_specs=[
            pl.BlockSpec((_BE, 32), lambda i: (i, 0)),
            pl.BlockSpec((32, _U), lambda i: (0, 0)),
            pl.BlockSpec((1, _U), lambda i: (0, 0)),
            pl.BlockSpec((_U, 2 * _U), lambda i: (0, 0)),
            pl.BlockSpec((1, 2 * _U), lambda i: (0, 0)),
            pl.BlockSpec((2 * _U, 3 * _U), lambda i: (0, 0)),
            pl.BlockSpec((1, 3 * _U), lambda i: (0, 0)),
        ],
        out_specs=pl.BlockSpec((_BE, 3 * _U), lambda i: (i, 0)),
        out_shape=jax.ShapeDtypeStruct((_E, 3 * _U), jnp.float32),
    )(ea, w1t, b1, w2t, b2, w3pt, b3p)


def _prep_body(x_ref, w0t_ref, w1t_ref, w2t_ref, xn_ref, v_ref):
    x = x_ref[...]
    c = [x[:, k * _U:(k + 1) * _U] for k in range(9)]
    norm = sum(e * e for e in c) + 1.0
    inv = 1.0 / norm
    c = [e * inv for e in c]
    xn_ref[...] = jnp.concatenate(c, axis=1)
    tr = (c[0] + c[4] + c[8]) * (1.0 / 3.0)
    a01 = 0.5 * (c[1] - c[3])
    a02 = 0.5 * (c[2] - c[6])
    a12 = 0.5 * (c[5] - c[7])
    s00 = c[0] - tr
    s01 = 0.5 * (c[1] + c[3])
    s02 = 0.5 * (c[2] + c[6])
    s11 = c[4] - tr
    s12 = 0.5 * (c[5] + c[7])
    s22 = c[8] - tr
    w0t, w1t, w2t = w0t_ref[...], w1t_ref[...], w2t_ref[...]

    def dot(a, b):
        return jnp.dot(a, b, preferred_element_type=jnp.float32)

    v_ref[...] = jnp.concatenate(
        [dot(tr, w0t),
         dot(a01, w1t), dot(a02, w1t), dot(a12, w1t),
         dot(s00, w2t), dot(s01, w2t), dot(s02, w2t),
         dot(s11, w2t), dot(s12, w2t), dot(s22, w2t)], axis=1)


def _node_prep(x9, w0t, w1t, w2t):
    grid = (_N // _BN,)
    return pl.pallas_call(
        _prep_body,
        grid=grid,
        in_specs=[
            pl.BlockSpec((_BN, 9 * _U), lambda i: (i, 0)),
            pl.BlockSpec((_U, _U), lambda i: (0, 0)),
            pl.BlockSpec((_U, _U), lambda i: (0, 0)),
            pl.BlockSpec((_U, _U), lambda i: (0, 0)),
        ],
        out_specs=[
            pl.BlockSpec((_BN, 9 * _U), lambda i: (i, 0)),
            pl.BlockSpec((_BN, 10 * _U), lambda i: (i, 0)),
        ],
        out_shape=[
            jax.ShapeDtypeStruct((_N, 9 * _U), jnp.float32),
            jax.ShapeDtypeStruct((_N, 10 * _U), jnp.float32),
        ],
    )(x9, w0t, w1t, w2t)


def _sc_body(v_hbm, src_hbm, rd_hbm, dst_hbm, cs_hbm, w_hbm, iptr_hbm,
             msg_hbm, lo_v, hi_v, dst_v, cs_v, idx_v, rd_v, rows_v, wrows_v,
             acc_v, sem1, sem2):
    wid = lax.axis_index("s") * 2 + lax.axis_index("c")
    n_lo = pl.multiple_of(wid * _NPW, 8)
    n_hi = pl.multiple_of(jnp.minimum(n_lo + _NPW, _N), 8)
    pltpu.sync_copy(iptr_hbm.at[pl.ds(n_lo, 16)], lo_v)
    pltpu.sync_copy(iptr_hbm.at[pl.ds(n_hi, 16)], hi_v)
    e_lo = lo_v[...][0]
    e_hi = hi_v[...][0]
    e0 = jnp.bitwise_and(e_lo, jnp.int32(-8))
    nch = (e_hi - e0 + (_K - 1)) // _K

    def load_small(ci, b):
        base = pl.multiple_of(e0 + ci * _K, 8)
        pltpu.sync_copy(src_hbm.at[pl.ds(base, _K)], idx_v.at[b])
        pltpu.sync_copy(rd_hbm.at[pl.ds(base, _K)], rd_v.at[b])
        pltpu.sync_copy(dst_hbm.at[pl.ds(base, _K)], dst_v.at[b, pl.ds(0, _K)])
        pltpu.sync_copy(cs_hbm.at[pl.ds(base, _K)], cs_v.at[b, pl.ds(0, _K)])

    def fire(b):
        pltpu.async_copy(v_hbm.at[idx_v.at[b]], rows_v.at[b], sem1)
        pltpu.async_copy(w_hbm.at[rd_v.at[b]], wrows_v.at[b], sem2)

    def drain(b):
        pltpu.make_async_copy(v_hbm.at[idx_v.at[b]], rows_v.at[b],
                              sem1).wait()
        pltpu.make_async_copy(w_hbm.at[rd_v.at[b]], wrows_v.at[b],
                              sem2).wait()

    @pl.when(nch > 0)
    def _():
        load_small(jnp.int32(0), jnp.int32(0))
        fire(jnp.int32(0))

    def chunk_body(ci, cd):
        par = jnp.bitwise_and(ci, 1)
        nxt = 1 - par
        drain(par)

        @pl.when(ci + 1 < nch)
        def _():
            load_small(ci + 1, nxt)
            fire(nxt)

        cdn = cd
        for p in range(2):
            accs = [acc_v[pl.ds(p * 640 + i * 16, 16)] for i in range(40)]

            def pass_body(j, carry, p=p, par=par):
                cdl = carry[0]
                accl = carry[1:]
                d = dst_v[par, pl.ds(j, 16)][0]
                valid = jnp.logical_and(d >= n_lo, d < n_hi)
                switch = jnp.logical_and(valid, d != cdl)

                @pl.when(jnp.logical_and(switch, cdl >= 0))
                def _():
                    for i in range(40):
                        acc_v[pl.ds(p * 640 + i * 16, 16)] = accl[i]
                    pltpu.sync_copy(acc_v.at[pl.ds(p * 640, 640)],
                                    msg_hbm.at[cdl, p])

                ce = cs_v[par, pl.ds(j, 16)][0]
                vf = jnp.where(valid, ce, 0.0)
                wcache = {}
                out = []
                for i in range(40):
                    s = p * 40 + i
                    comp = s // 8
                    u = s % 8
                    slot = 0 if comp == 0 else (1 if comp <= 3 else 2)
                    off = slot * _U + u * 16
                    if off not in wcache:
                        wcache[off] = wrows_v[par, j, pl.ds(off, 16)] * vf
                    base_a = jnp.where(switch, 0.0, accl[i])
                    out.append(base_a
                               + rows_v[par, j, pl.ds(s * 16, 16)]
                               * wcache[off])
                return (jnp.where(switch, d, cdl), *out)

            res = lax.fori_loop(0, _K, pass_body, (cdn, *accs))
            for i in range(40):
                acc_v[pl.ds(p * 640 + i * 16, 16)] = res[1 + i]
        cdn = res[0]
        return cdn

    cur = lax.fori_loop(0, nch, chunk_body, jnp.int32(-1))

    @pl.when(cur >= 0)
    def _():
        pltpu.sync_copy(acc_v.at[pl.ds(0, 640)], msg_hbm.at[cur, 0])
        pltpu.sync_copy(acc_v.at[pl.ds(640, 640)], msg_hbm.at[cur, 1])


@functools.cache
def _sc_segsum_built():
    return pl.kernel(
        _sc_body,
        out_type=jax.ShapeDtypeStruct((_N, 2, 640), jnp.float32),
        mesh=plsc.VectorSubcoreMesh(core_axis_name="c", subcore_axis_name="s"),
        scratch_types=[
            pltpu.VMEM((16,), jnp.int32),
            pltpu.VMEM((16,), jnp.int32),
            pltpu.VMEM((2, _K + 16), jnp.int32),
            pltpu.VMEM((2, _K + 16), jnp.float32),
            pltpu.VMEM((2, _K), jnp.int32),
            pltpu.VMEM((2, _K), jnp.int32),
            pltpu.VMEM((2, _K, 10 * _U), jnp.float32),
            pltpu.VMEM((2, _K, 3 * _U), jnp.float32),
            pltpu.VMEM((10 * _U,), jnp.float32),
            pltpu.SemaphoreType.DMA,
            pltpu.SemaphoreType.DMA,
        ],
    )


def _sc_segsum(*args):
    return _sc_segsum_built()(*args)


def _full9(t):
    iv, a01, a02, a12, s00, s01, s02, s11, s12, s22 = t
    return [iv + s00, a01 + s01, a02 + s02,
            s01 - a01, iv + s11, a12 + s12,
            s02 - a02, s12 - a12, iv + s22]


def _mm9(p, q):
    return [sum(p[3 * i + j] * q[3 * j + k] for j in range(3))
            for i in range(3) for k in range(3)]


def _fin_body(xn_ref, v_ref, msg_ref, deg_ref, w3t_ref, w4t_ref, w5t_ref,
              o_ref):
    xn = xn_ref[...]
    v = v_ref[...]
    m = msg_ref[...]
    degm = deg_ref[...] > 0.0
    vc = [v[:, k * _U:(k + 1) * _U] for k in range(10)]
    mc = [jnp.where(degm, m[:, k * _U:(k + 1) * _U], 0.0) for k in range(10)]
    y9 = _full9(vc)
    m9 = _full9(mc)
    ym = _mm9(y9, m9)
    my = _mm9(m9, y9)
    cm = [ym[t] + my[t] for t in range(9)]
    inv = 1.0 / (sum(e * e for e in cm) + 1.0)
    tr2 = (cm[0] + cm[4] + cm[8]) * (1.0 / 3.0)
    a01 = 0.5 * (cm[1] - cm[3]) * inv
    a02 = 0.5 * (cm[2] - cm[6]) * inv
    a12 = 0.5 * (cm[5] - cm[7]) * inv
    s00 = (cm[0] - tr2) * inv
    s01 = 0.5 * (cm[1] + cm[3]) * inv
    s02 = 0.5 * (cm[2] + cm[6]) * inv
    s11 = (cm[4] - tr2) * inv
    s12 = 0.5 * (cm[5] + cm[7]) * inv
    s22 = (cm[8] - tr2) * inv
    tr2 = tr2 * inv
    w3t, w4t, w5t = w3t_ref[...], w4t_ref[...], w5t_ref[...]

    def dot(a, b):
        return jnp.dot(a, b, preferred_element_type=jnp.float32)

    iv = dot(tr2, w3t)
    a01, a02, a12 = dot(a01, w4t), dot(a02, w4t), dot(a12, w4t)
    s00, s01, s02 = dot(s00, w5t), dot(s01, w5t), dot(s02, w5t)
    s11, s12, s22 = dot(s11, w5t), dot(s12, w5t), dot(s22, w5t)
    dx = [iv + s00, a01 + s01, a02 + s02,
          s01 - a01, iv + s11, a12 + s12,
          s02 - a02, s12 - a12, iv + s22]
    dxdx = _mm9(dx, dx)
    xnc = [xn[:, t * _U:(t + 1) * _U] for t in range(9)]
    o_ref[...] = jnp.concatenate(
        [xnc[t] + dx[t] + dxdx[t] for t in range(9)], axis=1)


def _finale(xn, v, msg, deg, w3t, w4t, w5t):
    grid = (_N // _BN,)
    return pl.pallas_call(
        _fin_body,
        grid=grid,
        in_specs=[
            pl.BlockSpec((_BN, 9 * _U), lambda i: (i, 0)),
            pl.BlockSpec((_BN, 10 * _U), lambda i: (i, 0)),
            pl.BlockSpec((_BN, 10 * _U), lambda i: (i, 0)),
            pl.BlockSpec((_BN, 1), lambda i: (i, 0)),
            pl.BlockSpec((_U, _U), lambda i: (0, 0)),
            pl.BlockSpec((_U, _U), lambda i: (0, 0)),
            pl.BlockSpec((_U, _U), lambda i: (0, 0)),
        ],
        out_specs=pl.BlockSpec((_BN, 9 * _U), lambda i: (i, 0)),
        out_shape=jax.ShapeDtypeStruct((_N, 9 * _U), jnp.float32),
    )(xn, v, msg, deg, w3t, w4t, w5t)


def kernel(X, edge_index, edge_weight, edge_attr, row_data, row_indices,
           row_indptr, col_data, col_indices, col_indptr,
           W_s1, b_s1, W_s2, b_s2, W_s3, b_s3,
           W_t0, W_t1, W_t2, W_t3, W_t4, W_t5):
    w3p = W_s3.reshape(_U, 3, 2 * _U).transpose(1, 0, 2).reshape(3 * _U,
                                                                 2 * _U)
    b3p = b_s3.reshape(_U, 3).T.reshape(1, 3 * _U)
    w_edges = _edge_mlp(edge_attr, W_s1.T, b_s1.reshape(1, _U), W_s2.T,
                        b_s2.reshape(1, 2 * _U), w3p.T, b3p)

    xn, v = _node_prep(X.reshape(_N, 9 * _U), W_t0.T, W_t1.T, W_t2.T)

    srcp = jnp.pad(row_indices, (0, _ESC - _E))
    rdp = jnp.pad(row_data, (0, _ESC - _E))
    dstp = jnp.pad(jnp.take(edge_index[0], row_data), (0, _ESC - _E),
                   constant_values=_N)
    iptr = jnp.pad(row_indptr, (0, 31), constant_values=_E)
    ews = jnp.pad(jnp.take(edge_weight, row_data), (0, 128)).reshape(
        (_E + 128) // 128, 128)
    csp = _cutoff(ews).reshape(_E + 128)
    msg = _sc_segsum(v, srcp, rdp, dstp, csp, w_edges, iptr)

    deg = (row_indptr[1:] - row_indptr[:-1]).astype(jnp.float32).reshape(
        _N, 1)
    out = _finale(xn, v, msg.reshape(_N, 10 * _U), deg, W_t3.T, W_t4.T,
                  W_t5.T)
    return out.reshape(_N, 3, 3, _U)


# speedup vs baseline: 1.0120x; 1.0120x over previous
import functools

import jax
import jax.numpy as jnp
from jax import lax
from jax.experimental import pallas as pl
from jax.experimental.pallas import tpu as pltpu
from jax.experimental.pallas import tpu_sc as plsc

_N = 10000
_E = 160000
_U = 128
_CUTOFF = 5.0

_BE = 640
_ESC = _E + 64
_BN = 400
_K = 32
_NPW = 320


def _cut_body(ew_ref, c_ref):
    r = ew_ref[...]
    c_ref[...] = (0.5 * (jnp.cos(jnp.pi * r / _CUTOFF) + 1.0)
                  * (r < _CUTOFF).astype(jnp.float32))


def _cutoff(ew2):
    return pl.pallas_call(
        _cut_body,
        out_shape=jax.ShapeDtypeStruct(ew2.shape, jnp.float32),
    )(ew2)


def _mlp_body(ea_ref, c_ref, w1t_ref, b1_ref, w2t_ref, b2_ref, w3pt_ref,
              b3p_ref, w_ref):
    ea = ea_ref[...]
    c = c_ref[...]
    h = jax.nn.silu(jnp.dot(ea, w1t_ref[...],
                            preferred_element_type=jnp.float32) + b1_ref[...])
    h = jax.nn.silu(jnp.dot(h, w2t_ref[...],
                            preferred_element_type=jnp.float32) + b2_ref[...])
    h = jax.nn.silu(jnp.dot(h, w3pt_ref[...],
                            preferred_element_type=jnp.float32) + b3p_ref[...])
    w_ref[...] = h * c


def _edge_mlp(ea, ew, w1t, b1, w2t, b2, w3pt, b3p):
    grid = (_E // _BE,)
    return pl.pallas_call(
        _mlp_body,
        grid=grid,
        in_specs=[
            pl.BlockSpec((_BE, 32), lambda i: (i, 0)),
            pl.BlockSpec((_BE, 1), lambda i: (i, 0)),
            pl.BlockSpec((32, _U), lambda i: (0, 0)),
            pl.BlockSpec((1, _U), lambda i: (0, 0)),
            pl.BlockSpec((_U, 2 * _U), lambda i: (0, 0)),
            pl.BlockSpec((1, 2 * _U), lambda i: (0, 0)),
            pl.BlockSpec((2 * _U, 3 * _U), lambda i: (0, 0)),
            pl.BlockSpec((1, 3 * _U), lambda i: (0, 0)),
        ],
        out_specs=pl.BlockSpec((_BE, 3 * _U), lambda i: (i, 0)),
        out_shape=jax.ShapeDtypeStruct((_E, 3 * _U), jnp.float32),
    )(ea, ew, w1t, b1, w2t, b2, w3pt, b3p)


def _prep_body(x_ref, w0t_ref, w1t_ref, w2t_ref, xn_ref, v_ref):
    x = x_ref[...]
    c = [x[:, k * _U:(k + 1) * _U] for k in range(9)]
    norm = sum(e * e for e in c) + 1.0
    inv = 1.0 / norm
    c = [e * inv for e in c]
    xn_ref[...] = jnp.concatenate(c, axis=1)
    tr = (c[0] + c[4] + c[8]) * (1.0 / 3.0)
    a01 = 0.5 * (c[1] - c[3])
    a02 = 0.5 * (c[2] - c[6])
    a12 = 0.5 * (c[5] - c[7])
    s00 = c[0] - tr
    s01 = 0.5 * (c[1] + c[3])
    s02 = 0.5 * (c[2] + c[6])
    s11 = c[4] - tr
    s12 = 0.5 * (c[5] + c[7])
    s22 = c[8] - tr
    w0t, w1t, w2t = w0t_ref[...], w1t_ref[...], w2t_ref[...]

    def dot(a, b):
        return jnp.dot(a, b, preferred_element_type=jnp.float32)

    v_ref[...] = jnp.concatenate(
        [dot(tr, w0t),
         dot(a01, w1t), dot(a02, w1t), dot(a12, w1t),
         dot(s00, w2t), dot(s01, w2t), dot(s02, w2t),
         dot(s11, w2t), dot(s12, w2t), dot(s22, w2t)], axis=1)


def _node_prep(x9, w0t, w1t, w2t):
    grid = (_N // _BN,)
    return pl.pallas_call(
        _prep_body,
        grid=grid,
        in_specs=[
            pl.BlockSpec((_BN, 9 * _U), lambda i: (i, 0)),
            pl.BlockSpec((_U, _U), lambda i: (0, 0)),
            pl.BlockSpec((_U, _U), lambda i: (0, 0)),
            pl.BlockSpec((_U, _U), lambda i: (0, 0)),
        ],
        out_specs=[
            pl.BlockSpec((_BN, 9 * _U), lambda i: (i, 0)),
            pl.BlockSpec((_BN, 10 * _U), lambda i: (i, 0)),
        ],
        out_shape=[
            jax.ShapeDtypeStruct((_N, 9 * _U), jnp.float32),
            jax.ShapeDtypeStruct((_N, 10 * _U), jnp.float32),
        ],
    )(x9, w0t, w1t, w2t)


def _sc_body(v_hbm, src_hbm, rd_hbm, dst_hbm, w_hbm, iptr_hbm, msg_hbm,
             lo_v, hi_v, dst_v, idx_v, rd_v, rows_v, wrows_v, acc_v,
             sem1, sem2):
    wid = lax.axis_index("s") * 2 + lax.axis_index("c")
    n_lo = pl.multiple_of(wid * _NPW, 8)
    n_hi = pl.multiple_of(jnp.minimum(n_lo + _NPW, _N), 8)
    pltpu.sync_copy(iptr_hbm.at[pl.ds(n_lo, 16)], lo_v)
    pltpu.sync_copy(iptr_hbm.at[pl.ds(n_hi, 16)], hi_v)
    e_lo = lo_v[...][0]
    e_hi = hi_v[...][0]
    e0 = jnp.bitwise_and(e_lo, jnp.int32(-8))
    nch = (e_hi - e0 + (_K - 1)) // _K

    def load_small(ci, b):
        base = pl.multiple_of(e0 + ci * _K, 8)
        pltpu.sync_copy(src_hbm.at[pl.ds(base, _K)], idx_v.at[b])
        pltpu.sync_copy(rd_hbm.at[pl.ds(base, _K)], rd_v.at[b])
        pltpu.sync_copy(dst_hbm.at[pl.ds(base, _K)], dst_v.at[b, pl.ds(0, _K)])

    def fire(b):
        pltpu.async_copy(v_hbm.at[idx_v.at[b]], rows_v.at[b], sem1)
        pltpu.async_copy(w_hbm.at[rd_v.at[b]], wrows_v.at[b], sem2)

    def drain(b):
        pltpu.make_async_copy(v_hbm.at[idx_v.at[b]], rows_v.at[b],
                              sem1).wait()
        pltpu.make_async_copy(w_hbm.at[rd_v.at[b]], wrows_v.at[b],
                              sem2).wait()

    @pl.when(nch > 0)
    def _():
        load_small(jnp.int32(0), jnp.int32(0))
        fire(jnp.int32(0))

    def chunk_body(ci, cd):
        par = jnp.bitwise_and(ci, 1)
        nxt = 1 - par
        drain(par)

        @pl.when(ci + 1 < nch)
        def _():
            load_small(ci + 1, nxt)
            fire(nxt)

        cdn = cd
        for p in range(2):
            accs = [acc_v[pl.ds(p * 640 + i * 16, 16)] for i in range(40)]

            def pass_body(j, carry, p=p, par=par):
                cdl = carry[0]
                accl = carry[1:]
                d = dst_v[par, pl.ds(j, 16)][0]
                valid = jnp.logical_and(d >= n_lo, d < n_hi)
                switch = jnp.logical_and(valid, d != cdl)

                @pl.when(jnp.logical_and(switch, cdl >= 0))
                def _():
                    for i in range(40):
                        acc_v[pl.ds(p * 640 + i * 16, 16)] = accl[i]
                    pltpu.sync_copy(acc_v.at[pl.ds(p * 640, 640)],
                                    msg_hbm.at[cdl, p])

                vf = jnp.where(valid, 1.0, 0.0)
                wcache = {}
                out = []
                for i in range(40):
                    s = p * 40 + i
                    comp = s // 8
                    u = s % 8
                    slot = 0 if comp == 0 else (1 if comp <= 3 else 2)
                    off = slot * _U + u * 16
                    if off not in wcache:
                        wcache[off] = wrows_v[par, j, pl.ds(off, 16)] * vf
                    base_a = jnp.where(switch, 0.0, accl[i])
                    out.append(base_a
                               + rows_v[par, j, pl.ds(s * 16, 16)]
                               * wcache[off])
                return (jnp.where(switch, d, cdl), *out)

            res = lax.fori_loop(0, _K, pass_body, (cdn, *accs))
            for i in range(40):
                acc_v[pl.ds(p * 640 + i * 16, 16)] = res[1 + i]
        cdn = res[0]
        return cdn

    cur = lax.fori_loop(0, nch, chunk_body, jnp.int32(-1))

    @pl.when(cur >= 0)
    def _():
        pltpu.sync_copy(acc_v.at[pl.ds(0, 640)], msg_hbm.at[cur, 0])
        pltpu.sync_copy(acc_v.at[pl.ds(640, 640)], msg_hbm.at[cur, 1])


@functools.cache
def _sc_segsum_built():
    return pl.kernel(
        _sc_body,
        out_type=jax.ShapeDtypeStruct((_N, 2, 640), jnp.float32),
        mesh=plsc.VectorSubcoreMesh(core_axis_name="c", subcore_axis_name="s"),
        scratch_types=[
            pltpu.VMEM((16,), jnp.int32),
            pltpu.VMEM((16,), jnp.int32),
            pltpu.VMEM((2, _K + 16), jnp.int32),
            pltpu.VMEM((2, _K), jnp.int32),
            pltpu.VMEM((2, _K), jnp.int32),
            pltpu.VMEM((2, _K, 10 * _U), jnp.float32),
            pltpu.VMEM((2, _K, 3 * _U), jnp.float32),
            pltpu.VMEM((10 * _U,), jnp.float32),
            pltpu.SemaphoreType.DMA,
            pltpu.SemaphoreType.DMA,
        ],
    )


def _sc_segsum(*args):
    return _sc_segsum_built()(*args)


def _full9(t):
    iv, a01, a02, a12, s00, s01, s02, s11, s12, s22 = t
    return [iv + s00, a01 + s01, a02 + s02,
            s01 - a01, iv + s11, a12 + s12,
            s02 - a02, s12 - a12, iv + s22]


def _mm9(p, q):
    return [sum(p[3 * i + j] * q[3 * j + k] for j in range(3))
            for i in range(3) for k in range(3)]


def _fin_body(xn_ref, v_ref, msg_ref, deg_ref, w3t_ref, w4t_ref, w5t_ref,
              o_ref):
    xn = xn_ref[...]
    v = v_ref[...]
    m = msg_ref[...]
    degm = deg_ref[...] > 0.0
    vc = [v[:, k * _U:(k + 1) * _U] for k in range(10)]
    mc = [jnp.where(degm, m[:, k * _U:(k + 1) * _U], 0.0) for k in range(10)]
    y9 = _full9(vc)
    m9 = _full9(mc)
    ym = _mm9(y9, m9)
    my = _mm9(m9, y9)
    cm = [ym[t] + my[t] for t in range(9)]
    inv = 1.0 / (sum(e * e for e in cm) + 1.0)
    tr2 = (cm[0] + cm[4] + cm[8]) * (1.0 / 3.0)
    a01 = 0.5 * (cm[1] - cm[3]) * inv
    a02 = 0.5 * (cm[2] - cm[6]) * inv
    a12 = 0.5 * (cm[5] - cm[7]) * inv
    s00 = (cm[0] - tr2) * inv
    s01 = 0.5 * (cm[1] + cm[3]) * inv
    s02 = 0.5 * (cm[2] + cm[6]) * inv
    s11 = (cm[4] - tr2) * inv
    s12 = 0.5 * (cm[5] + cm[7]) * inv
    s22 = (cm[8] - tr2) * inv
    tr2 = tr2 * inv
    w3t, w4t, w5t = w3t_ref[...], w4t_ref[...], w5t_ref[...]

    def dot(a, b):
        return jnp.dot(a, b, preferred_element_type=jnp.float32)

    iv = dot(tr2, w3t)
    a01, a02, a12 = dot(a01, w4t), dot(a02, w4t), dot(a12, w4t)
    s00, s01, s02 = dot(s00, w5t), dot(s01, w5t), dot(s02, w5t)
    s11, s12, s22 = dot(s11, w5t), dot(s12, w5t), dot(s22, w5t)
    dx = [iv + s00, a01 + s01, a02 + s02,
          s01 - a01, iv + s11, a12 + s12,
          s02 - a02, s12 - a12, iv + s22]
    dxdx = _mm9(dx, dx)
    xnc = [xn[:, t * _U:(t + 1) * _U] for t in range(9)]
    o_ref[...] = jnp.concatenate(
        [xnc[t] + dx[t] + dxdx[t] for t in range(9)], axis=1)


def _finale(xn, v, msg, deg, w3t, w4t, w5t):
    grid = (_N // _BN,)
    return pl.pallas_call(
        _fin_body,
        grid=grid,
        in_specs=[
            pl.BlockSpec((_BN, 9 * _U), lambda i: (i, 0)),
            pl.BlockSpec((_BN, 10 * _U), lambda i: (i, 0)),
            pl.BlockSpec((_BN, 10 * _U), lambda i: (i, 0)),
            pl.BlockSpec((_BN, 1), lambda i: (i, 0)),
            pl.BlockSpec((_U, _U), lambda i: (0, 0)),
            pl.BlockSpec((_U, _U), lambda i: (0, 0)),
            pl.BlockSpec((_U, _U), lambda i: (0, 0)),
        ],
        out_specs=pl.BlockSpec((_BN, 9 * _U), lambda i: (i, 0)),
        out_shape=jax.ShapeDtypeStruct((_N, 9 * _U), jnp.float32),
    )(xn, v, msg, deg, w3t, w4t, w5t)


def kernel(X, edge_index, edge_weight, edge_attr, row_data, row_indices,
           row_indptr, col_data, col_indices, col_indptr,
           W_s1, b_s1, W_s2, b_s2, W_s3, b_s3,
           W_t0, W_t1, W_t2, W_t3, W_t4, W_t5):
    w3p = W_s3.reshape(_U, 3, 2 * _U).transpose(1, 0, 2).reshape(3 * _U,
                                                                 2 * _U)
    b3p = b_s3.reshape(_U, 3).T.reshape(1, 3 * _U)
    ew2 = edge_weight.reshape(_E // 128, 128)
    c_col = _cutoff(ew2).reshape(_E, 1)
    w_edges = _edge_mlp(edge_attr, c_col, W_s1.T, b_s1.reshape(1, _U), W_s2.T,
                        b_s2.reshape(1, 2 * _U), w3p.T, b3p)

    xn, v = _node_prep(X.reshape(_N, 9 * _U), W_t0.T, W_t1.T, W_t2.T)

    srcp = jnp.pad(row_indices, (0, _ESC - _E))
    rdp = jnp.pad(row_data, (0, _ESC - _E))
    dstp = jnp.pad(jnp.take(edge_index[0], row_data), (0, _ESC - _E),
                   constant_values=_N)
    iptr = jnp.pad(row_indptr, (0, 31), constant_values=_E)
    msg = _sc_segsum(v, srcp, rdp, dstp, w_edges, iptr)

    deg = (row_indptr[1:] - row_indptr[:-1]).astype(jnp.float32).reshape(
        _N, 1)
    out = _finale(xn, v, msg.reshape(_N, 10 * _U), deg, W_t3.T, W_t4.T,
                  W_t5.T)
    return out.reshape(_N, 3, 3, _U)
